# baseline (speedup 1.0000x reference)
import functools

import jax
import jax.numpy as jnp
from jax import lax
from jax.experimental import pallas as pl
from jax.experimental.pallas import tpu as pltpu
from jax.experimental.pallas import tpu_sc as plsc

N_NODES = 10000
N_EDGES = 320000
D_FEAT = 128
D_HID = 16

NC = 2
NS = 16
NW = NC * NS
L = 16

NP = 10240
RPT = NP // NS
CHUNK = 128
NCHUNKS = 2560
E_PAD = NCHUNKS * CHUNK
Q0 = 96
Q1 = 64
QMAX = max(Q0, Q1)
NB = 4

_mesh = plsc.VectorSubcoreMesh(core_axis_name="c", subcore_axis_name="s")


def _zero_fill(zbuf, acc, sid):
    for r in range(CHUNK):
        zbuf[r, :] = jnp.zeros((L,), jnp.float32)

    def body(j, _):
        pltpu.sync_copy(zbuf, acc.at[pl.ds(sid * RPT + j * CHUNK, CHUNK)])
        return 0

    lax.fori_loop(0, RPT // CHUNK, body, 0)


def _copy_out(acc, out_hbm, cid, sid):
    pltpu.sync_copy(acc.at[pl.ds(sid * RPT, RPT)],
                    out_hbm.at[cid, pl.ds(sid * RPT, RPT)])


@functools.partial(
    pl.kernel,
    out_type=jax.ShapeDtypeStruct((NC, NP, L), jnp.float32),
    mesh=_mesh,
    compiler_params=pltpu.CompilerParams(use_tc_tiling_on_sc=False),
    scratch_types=[
        pltpu.VMEM((QMAX, CHUNK), jnp.int32),
        pltpu.VMEM((QMAX, CHUNK), jnp.int32),
        [pltpu.VMEM((CHUNK, L), jnp.float32)] * NB,
        pltpu.VMEM_SHARED((NP, L), jnp.float32),
        [pltpu.SemaphoreType.DMA] * NB,
        [pltpu.SemaphoreType.DMA] * NB,
    ],
)
def _sc_scatter(y_hbm, row_hbm, col_hbm, out_hbm,
                ridx, cidx, rows, acc, gsems, ssems):
    cid = lax.axis_index("c")
    sid = lax.axis_index("s")
    _zero_fill(rows[0], acc, sid)

    def _edge_loop(q, start):
        pltpu.sync_copy(row_hbm.at[pl.ds(start, q)], ridx.at[pl.ds(0, q)])
        pltpu.sync_copy(col_hbm.at[pl.ds(start, q)], cidx.at[pl.ds(0, q)])
        plsc.subcore_barrier()

        for j in range(NB - 1):
            pltpu.async_copy(y_hbm.at[ridx.at[j]], rows[j], gsems[j])

        @pl.loop(0, q // NB)
        def _steps(g):
            i0 = g * NB
            for b in range(NB):
                i = i0 + b
                pltpu.make_async_copy(y_hbm.at[ridx.at[i]], rows[b],
                                      gsems[b]).wait()

                pb = (b - 1) % NB
                @pl.when(i > 0)
                def _drain_prev():
                    pltpu.make_async_copy(rows[pb], acc.at[cidx.at[i]],
                                          ssems[pb]).wait()

                @pl.when(i + NB - 1 < q)
                def _prefetch():
                    pltpu.async_copy(y_hbm.at[ridx.at[i + NB - 1]], rows[pb],
                                     gsems[pb])

                pltpu.async_copy(rows[b], acc.at[cidx.at[i]], ssems[b],
                                 add=True)

        pltpu.make_async_copy(rows[NB - 1], acc.at[cidx.at[q - 1]],
                              ssems[NB - 1]).wait()

    @pl.when(cid == 0)
    def _sc0():
        _edge_loop(Q0, sid * Q0)

    @pl.when(cid == 1)
    def _sc1():
        _edge_loop(Q1, NS * Q0 + sid * Q1)

    plsc.subcore_barrier()
    _copy_out(acc, out_hbm, cid, sid)


@functools.partial(
    pl.kernel,
    out_type=jax.ShapeDtypeStruct((NC, NP, L), jnp.float32),
    mesh=_mesh,
    compiler_params=pltpu.CompilerParams(use_tc_tiling_on_sc=False),
    scratch_types=[
        pltpu.VMEM((QMAX, CHUNK), jnp.int32),
        pltpu.VMEM((CHUNK, L), jnp.float32),
        pltpu.VMEM_SHARED((NP, L), jnp.float32),
        pltpu.SemaphoreType.DMA,
    ],
)
def _sc_degree(col_hbm, out_hbm, cidx, ones, acc, sem):
    cid = lax.axis_index("c")
    sid = lax.axis_index("s")
    _zero_fill(ones, acc, sid)
    for r in range(CHUNK):
        ones[r, :] = jnp.ones((L,), jnp.float32)

    WIN = 8

    def _deg_loop(q, start):
        pltpu.sync_copy(col_hbm.at[pl.ds(start, q)], cidx.at[pl.ds(0, q)])
        plsc.subcore_barrier()

        @pl.loop(0, q)
        def _steps(i):
            pltpu.async_copy(ones, acc.at[cidx.at[i]], sem, add=True)

            @pl.when(i >= WIN)
            def _roll():
                pltpu.make_async_copy(ones, acc.at[cidx.at[i]], sem).wait()

        @pl.loop(0, WIN)
        def _drain(i):
            pltpu.make_async_copy(ones, acc.at[cidx.at[0]], sem).wait()

    @pl.when(cid == 0)
    def _sc0():
        _deg_loop(Q0, sid * Q0)

    @pl.when(cid == 1)
    def _sc1():
        _deg_loop(Q1, NS * Q0 + sid * Q1)

    plsc.subcore_barrier()
    _copy_out(acc, out_hbm, cid, sid)


def _tc_mm_body(x_ref, w0_ref, w1_ref, y0_ref, y1_ref):
    x = x_ref[...]
    y0_ref[...] = jnp.dot(x, w0_ref[...], preferred_element_type=jnp.float32)
    y1_ref[...] = jnp.dot(x, w1_ref[...], preferred_element_type=jnp.float32)


def _tc_scale_body(y1_ref, d_ref, y1s_ref, dis_ref):
    deg = d_ref[0, :N_NODES, :] + d_ref[1, :N_NODES, :]
    dis = jnp.where(deg > 0.0, lax.rsqrt(deg), 0.0)
    dis_ref[...] = dis
    y1s_ref[pl.ds(0, N_NODES), :] = dis * y1_ref[...]
    y1s_ref[pl.ds(N_NODES, NP - N_NODES), :] = jnp.zeros(
        (NP - N_NODES, D_HID), jnp.float32)


def _tc2_body(y0_ref, a_ref, dis_ref, b1_ref, w0_ref, w1_ref,
              z0_ref, z1s_ref):
    dis = dis_ref[...]
    agg = a_ref[0, :N_NODES, :] + a_ref[1, :N_NODES, :]
    h = y0_ref[...] + dis * agg + b1_ref[...]
    h = jnp.maximum(h, 0.0)
    z0_ref[...] = jnp.dot(h, w0_ref[...], preferred_element_type=jnp.float32)
    z1 = jnp.dot(h, w1_ref[...], preferred_element_type=jnp.float32)
    z1s_ref[pl.ds(0, N_NODES), :] = dis * z1
    z1s_ref[pl.ds(N_NODES, NP - N_NODES), :] = jnp.zeros(
        (NP - N_NODES, D_HID), jnp.float32)


def _tc3_body(z0_ref, a_ref, dis_ref, b2_ref, out_ref):
    agg = a_ref[0, :N_NODES, :] + a_ref[1, :N_NODES, :]
    o = z0_ref[...] + dis_ref[...] * agg + b2_ref[...]
    m = jnp.max(o, axis=1, keepdims=True)
    s = jnp.sum(jnp.exp(o - m), axis=1, keepdims=True)
    out_ref[...] = o - m - jnp.log(s)


def kernel(x, edge_index, W1_0, W1_1, b1, W2_0, W2_1, b2):
    n = x.shape[0]
    row = edge_index[0].astype(jnp.int32)
    col = edge_index[1].astype(jnp.int32)
    npad = E_PAD - N_EDGES
    row3 = jnp.concatenate(
        [row, jnp.zeros((npad,), jnp.int32)]).reshape(NCHUNKS, CHUNK)
    col3 = jnp.concatenate(
        [col, jnp.full((npad,), NP - 1, jnp.int32)]).reshape(NCHUNKS, CHUNK)

    degp = _sc_degree(col3)

    y0, y1 = pl.pallas_call(
        _tc_mm_body,
        out_shape=(
            jax.ShapeDtypeStruct((n, D_HID), jnp.float32),
            jax.ShapeDtypeStruct((n, D_HID), jnp.float32),
        ),
    )(x, W1_0, W1_1)

    y1s_p, dis = pl.pallas_call(
        _tc_scale_body,
        out_shape=(
            jax.ShapeDtypeStruct((NP, D_HID), jnp.float32),
            jax.ShapeDtypeStruct((n, D_HID), jnp.float32),
        ),
    )(y1, degp)

    agg1 = _sc_scatter(y1s_p, row3, col3)

    z0, z1s_p = pl.pallas_call(
        _tc2_body,
        out_shape=(
            jax.ShapeDtypeStruct((n, D_HID), jnp.float32),
            jax.ShapeDtypeStruct((NP, D_HID), jnp.float32),
        ),
    )(y0, agg1, dis, b1.reshape(1, D_HID), W2_0, W2_1)

    agg2 = _sc_scatter(z1s_p, row3, col3)

    out = pl.pallas_call(
        _tc3_body,
        out_shape=jax.ShapeDtypeStruct((n, D_HID), jnp.float32),
    )(z0, agg2, dis, b2.reshape(1, D_HID))
    return out


# speedup vs baseline: 1.2712x; 1.2712x over previous
import functools

import jax
import jax.numpy as jnp
from jax import lax
from jax.experimental import pallas as pl
from jax.experimental.pallas import tpu as pltpu
from jax.experimental.pallas import tpu_sc as plsc

N_NODES = 10000
N_EDGES = 320000
D_FEAT = 128
D_HID = 16

NC = 2
NS = 16
NW = NC * NS
L = 16

NP = 10240
RPT = NP // NS
CHUNK = 128
NCHUNKS = N_EDGES // CHUNK
Q0 = 104
Q1 = 52
TAIL_BASE = NS * (Q0 + Q1)
QMAX = max(Q0, Q1)
NB = 4

_mesh = plsc.VectorSubcoreMesh(core_axis_name="c", subcore_axis_name="s")


def _zero_fill(zbuf, acc, sid):
    for r in range(CHUNK):
        zbuf[r, :] = jnp.zeros((L,), jnp.float32)

    def body(j, _):
        pltpu.sync_copy(zbuf, acc.at[pl.ds(sid * RPT + j * CHUNK, CHUNK)])
        return 0

    lax.fori_loop(0, RPT // CHUNK, body, 0)


def _copy_out(acc, out_hbm, cid, sid):
    pltpu.sync_copy(acc.at[pl.ds(sid * RPT, RPT)],
                    out_hbm.at[cid, pl.ds(sid * RPT, RPT)])


@functools.partial(
    pl.kernel,
    out_type=jax.ShapeDtypeStruct((NC, NP, L), jnp.float32),
    mesh=_mesh,
    compiler_params=pltpu.CompilerParams(use_tc_tiling_on_sc=False),
    scratch_types=[
        pltpu.VMEM((QMAX, CHUNK), jnp.int32),
        pltpu.VMEM((QMAX, CHUNK), jnp.int32),
        [pltpu.VMEM((CHUNK, L), jnp.float32)] * NB,
        pltpu.VMEM((CHUNK,), jnp.int32),
        pltpu.VMEM((CHUNK,), jnp.int32),
        pltpu.VMEM_SHARED((NP, L), jnp.float32),
        [pltpu.SemaphoreType.DMA] * NB,
        [pltpu.SemaphoreType.DMA] * NB,
    ],
)
def _sc_scatter(y_hbm, row_hbm, col_hbm, out_hbm,
                ridx, cidx, rows, trow, tcol, acc, gsems, ssems):
    cid = lax.axis_index("c")
    sid = lax.axis_index("s")
    _zero_fill(rows[0], acc, sid)

    def _edge_loop(q, start):
        pltpu.sync_copy(row_hbm.at[pl.ds(start, q)], ridx.at[pl.ds(0, q)])
        pltpu.sync_copy(col_hbm.at[pl.ds(start, q)], cidx.at[pl.ds(0, q)])
        plsc.subcore_barrier()

        for j in range(NB - 1):
            pltpu.async_copy(y_hbm.at[ridx.at[j]], rows[j], gsems[j])

        @pl.loop(0, q // NB)
        def _steps(g):
            i0 = g * NB
            for b in range(NB):
                i = i0 + b
                pltpu.make_async_copy(y_hbm.at[ridx.at[i]], rows[b],
                                      gsems[b]).wait()

                pb = (b - 1) % NB
                @pl.when(i > 0)
                def _drain_prev():
                    pltpu.make_async_copy(rows[pb], acc.at[cidx.at[i]],
                                          ssems[pb]).wait()

                @pl.when(i + NB - 1 < q)
                def _prefetch():
                    pltpu.async_copy(y_hbm.at[ridx.at[i + NB - 1]], rows[pb],
                                     gsems[pb])

                pltpu.async_copy(rows[b], acc.at[cidx.at[i]], ssems[b],
                                 add=True)

        pltpu.make_async_copy(rows[NB - 1], acc.at[cidx.at[q - 1]],
                              ssems[NB - 1]).wait()

    @pl.when(cid == 0)
    def _sc0():
        _edge_loop(Q0, sid * Q0)

    @pl.when(cid == 1)
    def _sc1():
        _edge_loop(Q1, NS * Q0 + sid * Q1)

    @pl.when(jnp.logical_and(cid == 1, sid >= NS - (NCHUNKS - TAIL_BASE)))
    def _tail():
        tc = TAIL_BASE + sid - (NS - (NCHUNKS - TAIL_BASE))
        pltpu.sync_copy(row_hbm.at[tc], trow)
        pltpu.sync_copy(col_hbm.at[tc], tcol)
        pltpu.async_copy(y_hbm.at[trow], rows[0], gsems[0]).wait()
        pltpu.sync_copy(rows[0], acc.at[tcol], add=True)

    plsc.subcore_barrier()
    _copy_out(acc, out_hbm, cid, sid)


@functools.partial(
    pl.kernel,
    out_type=jax.ShapeDtypeStruct((NC, NP, L), jnp.float32),
    mesh=_mesh,
    compiler_params=pltpu.CompilerParams(use_tc_tiling_on_sc=False),
    scratch_types=[
        pltpu.VMEM((QMAX, CHUNK), jnp.int32),
        pltpu.VMEM((CHUNK, L), jnp.float32),
        pltpu.VMEM((CHUNK,), jnp.int32),
        pltpu.VMEM_SHARED((NP, L), jnp.float32),
        pltpu.SemaphoreType.DMA,
    ],
)
def _sc_degree(col_hbm, out_hbm, cidx, ones, tcol, acc, sem):
    cid = lax.axis_index("c")
    sid = lax.axis_index("s")
    _zero_fill(ones, acc, sid)
    for r in range(CHUNK):
        ones[r, :] = jnp.ones((L,), jnp.float32)

    WIN = 8

    def _deg_loop(q, start):
        pltpu.sync_copy(col_hbm.at[pl.ds(start, q)], cidx.at[pl.ds(0, q)])
        plsc.subcore_barrier()

        @pl.loop(0, q)
        def _steps(i):
            pltpu.async_copy(ones, acc.at[cidx.at[i]], sem, add=True)

            @pl.when(i >= WIN)
            def _roll():
                pltpu.make_async_copy(ones, acc.at[cidx.at[i]], sem).wait()

        @pl.loop(0, WIN)
        def _drain(i):
            pltpu.make_async_copy(ones, acc.at[cidx.at[0]], sem).wait()

    @pl.when(cid == 0)
    def _sc0():
        _deg_loop(Q0, sid * Q0)

    @pl.when(cid == 1)
    def _sc1():
        _deg_loop(Q1, NS * Q0 + sid * Q1)

    @pl.when(jnp.logical_and(cid == 1, sid >= NS - (NCHUNKS - TAIL_BASE)))
    def _tail():
        tc = TAIL_BASE + sid - (NS - (NCHUNKS - TAIL_BASE))
        pltpu.sync_copy(col_hbm.at[tc], tcol)
        pltpu.sync_copy(ones, acc.at[tcol], add=True)

    plsc.subcore_barrier()
    _copy_out(acc, out_hbm, cid, sid)


def _tc_mm_body(x_ref, w0_ref, w1_ref, y0_ref, y1_ref):
    x = x_ref[...]
    y0_ref[...] = jnp.dot(x, w0_ref[...], preferred_element_type=jnp.float32)
    y1_ref[...] = jnp.dot(x, w1_ref[...], preferred_element_type=jnp.float32)


def _tc_scale_body(y1_ref, d_ref, y1s_ref, dis_ref):
    deg = d_ref[0, :N_NODES, :] + d_ref[1, :N_NODES, :]
    dis = jnp.where(deg > 0.0, lax.rsqrt(deg), 0.0)
    dis_ref[...] = dis
    y1s_ref[pl.ds(0, N_NODES), :] = dis * y1_ref[...]
    y1s_ref[pl.ds(N_NODES, NP - N_NODES), :] = jnp.zeros(
        (NP - N_NODES, D_HID), jnp.float32)


def _tc2_body(y0_ref, a_ref, dis_ref, b1_ref, w0_ref, w1_ref,
              z0_ref, z1s_ref):
    dis = dis_ref[...]
    agg = a_ref[0, :N_NODES, :] + a_ref[1, :N_NODES, :]
    h = y0_ref[...] + dis * agg + b1_ref[...]
    h = jnp.maximum(h, 0.0)
    z0_ref[...] = jnp.dot(h, w0_ref[...], preferred_element_type=jnp.float32)
    z1 = jnp.dot(h, w1_ref[...], preferred_element_type=jnp.float32)
    z1s_ref[pl.ds(0, N_NODES), :] = dis * z1
    z1s_ref[pl.ds(N_NODES, NP - N_NODES), :] = jnp.zeros(
        (NP - N_NODES, D_HID), jnp.float32)


def _tc3_body(z0_ref, a_ref, dis_ref, b2_ref, out_ref):
    agg = a_ref[0, :N_NODES, :] + a_ref[1, :N_NODES, :]
    o = z0_ref[...] + dis_ref[...] * agg + b2_ref[...]
    m = jnp.max(o, axis=1, keepdims=True)
    s = jnp.sum(jnp.exp(o - m), axis=1, keepdims=True)
    out_ref[...] = o - m - jnp.log(s)


def kernel(x, edge_index, W1_0, W1_1, b1, W2_0, W2_1, b2):
    n = x.shape[0]
    row3 = edge_index[0].astype(jnp.int32).reshape(NCHUNKS, CHUNK)
    col3 = edge_index[1].astype(jnp.int32).reshape(NCHUNKS, CHUNK)

    degp = _sc_degree(col3)

    y0, y1 = pl.pallas_call(
        _tc_mm_body,
        out_shape=(
            jax.ShapeDtypeStruct((n, D_HID), jnp.float32),
            jax.ShapeDtypeStruct((n, D_HID), jnp.float32),
        ),
    )(x, W1_0, W1_1)

    y1s_p, dis = pl.pallas_call(
        _tc_scale_body,
        out_shape=(
            jax.ShapeDtypeStruct((NP, D_HID), jnp.float32),
            jax.ShapeDtypeStruct((n, D_HID), jnp.float32),
        ),
    )(y1, degp)

    agg1 = _sc_scatter(y1s_p, row3, col3)

    z0, z1s_p = pl.pallas_call(
        _tc2_body,
        out_shape=(
            jax.ShapeDtypeStruct((n, D_HID), jnp.float32),
            jax.ShapeDtypeStruct((NP, D_HID), jnp.float32),
        ),
    )(y0, agg1, dis, b1.reshape(1, D_HID), W2_0, W2_1)

    agg2 = _sc_scatter(z1s_p, row3, col3)

    out = pl.pallas_call(
        _tc3_body,
        out_shape=jax.ShapeDtypeStruct((n, D_HID), jnp.float32),
    )(z0, agg2, dis, b2.reshape(1, D_HID))
    return out


# speedup vs baseline: 1.2857x; 1.0114x over previous
import functools

import jax
import jax.numpy as jnp
from jax import lax
from jax.experimental import pallas as pl
from jax.experimental.pallas import tpu as pltpu
from jax.experimental.pallas import tpu_sc as plsc

N_NODES = 10000
N_EDGES = 320000
D_FEAT = 128
D_HID = 16

NC = 2
NS = 16
NW = NC * NS
L = 16

NP = 10240
RPT = NP // NS
CHUNK = 128
NCHUNKS = N_EDGES // CHUNK
Q0 = 100
Q1 = 56
TAIL_BASE = NS * (Q0 + Q1)
QMAX = max(Q0, Q1)
NB = 4

_mesh = plsc.VectorSubcoreMesh(core_axis_name="c", subcore_axis_name="s")


def _zero_fill(zbuf, acc, sid):
    for r in range(CHUNK):
        zbuf[r, :] = jnp.zeros((L,), jnp.float32)

    def body(j, _):
        pltpu.sync_copy(zbuf, acc.at[pl.ds(sid * RPT + j * CHUNK, CHUNK)])
        return 0

    lax.fori_loop(0, RPT // CHUNK, body, 0)


def _copy_out(acc, out_hbm, cid, sid):
    pltpu.sync_copy(acc.at[pl.ds(sid * RPT, RPT)],
                    out_hbm.at[cid, pl.ds(sid * RPT, RPT)])


@functools.partial(
    pl.kernel,
    out_type=jax.ShapeDtypeStruct((NC, NP, L), jnp.float32),
    mesh=_mesh,
    compiler_params=pltpu.CompilerParams(use_tc_tiling_on_sc=False),
    scratch_types=[
        pltpu.VMEM((QMAX, CHUNK), jnp.int32),
        pltpu.VMEM((QMAX, CHUNK), jnp.int32),
        [pltpu.VMEM((CHUNK, L), jnp.float32)] * NB,
        pltpu.VMEM((CHUNK,), jnp.int32),
        pltpu.VMEM((CHUNK,), jnp.int32),
        pltpu.VMEM_SHARED((NP, L), jnp.float32),
        [pltpu.SemaphoreType.DMA] * NB,
        [pltpu.SemaphoreType.DMA] * NB,
    ],
)
def _sc_scatter(y_hbm, row_hbm, col_hbm, out_hbm,
                ridx, cidx, rows, trow, tcol, acc, gsems, ssems):
    cid = lax.axis_index("c")
    sid = lax.axis_index("s")
    _zero_fill(rows[0], acc, sid)

    def _edge_loop(q, start):
        pltpu.sync_copy(row_hbm.at[pl.ds(start, q)], ridx.at[pl.ds(0, q)])
        pltpu.sync_copy(col_hbm.at[pl.ds(start, q)], cidx.at[pl.ds(0, q)])
        plsc.subcore_barrier()

        for j in range(NB - 1):
            pltpu.async_copy(y_hbm.at[ridx.at[j]], rows[j], gsems[j])

        @pl.loop(0, q // NB)
        def _steps(g):
            i0 = g * NB
            for b in range(NB):
                i = i0 + b
                pltpu.make_async_copy(y_hbm.at[ridx.at[i]], rows[b],
                                      gsems[b]).wait()

                pb = (b - 1) % NB
                @pl.when(i > 0)
                def _drain_prev():
                    pltpu.make_async_copy(rows[pb], acc.at[cidx.at[i]],
                                          ssems[pb]).wait()

                @pl.when(i + NB - 1 < q)
                def _prefetch():
                    pltpu.async_copy(y_hbm.at[ridx.at[i + NB - 1]], rows[pb],
                                     gsems[pb])

                pltpu.async_copy(rows[b], acc.at[cidx.at[i]], ssems[b],
                                 add=True)

        pltpu.make_async_copy(rows[NB - 1], acc.at[cidx.at[q - 1]],
                              ssems[NB - 1]).wait()

    @pl.when(cid == 0)
    def _sc0():
        _edge_loop(Q0, sid * Q0)

    @pl.when(cid == 1)
    def _sc1():
        _edge_loop(Q1, NS * Q0 + sid * Q1)

    @pl.when(jnp.logical_and(cid == 1, sid >= NS - (NCHUNKS - TAIL_BASE)))
    def _tail():
        tc = TAIL_BASE + sid - (NS - (NCHUNKS - TAIL_BASE))
        pltpu.sync_copy(row_hbm.at[tc], trow)
        pltpu.sync_copy(col_hbm.at[tc], tcol)
        pltpu.async_copy(y_hbm.at[trow], rows[0], gsems[0]).wait()
        pltpu.sync_copy(rows[0], acc.at[tcol], add=True)

    plsc.subcore_barrier()
    _copy_out(acc, out_hbm, cid, sid)


@functools.partial(
    pl.kernel,
    out_type=jax.ShapeDtypeStruct((NC, NP, L), jnp.float32),
    mesh=_mesh,
    compiler_params=pltpu.CompilerParams(use_tc_tiling_on_sc=False),
    scratch_types=[
        pltpu.VMEM((QMAX, CHUNK), jnp.int32),
        pltpu.VMEM((CHUNK, L), jnp.float32),
        pltpu.VMEM((CHUNK,), jnp.int32),
        pltpu.VMEM_SHARED((NP, L), jnp.float32),
        pltpu.SemaphoreType.DMA,
    ],
)
def _sc_degree(col_hbm, out_hbm, cidx, ones, tcol, acc, sem):
    cid = lax.axis_index("c")
    sid = lax.axis_index("s")
    _zero_fill(ones, acc, sid)
    for r in range(CHUNK):
        ones[r, :] = jnp.ones((L,), jnp.float32)

    WIN = 8

    def _deg_loop(q, start):
        pltpu.sync_copy(col_hbm.at[pl.ds(start, q)], cidx.at[pl.ds(0, q)])
        plsc.subcore_barrier()

        @pl.loop(0, q)
        def _steps(i):
            pltpu.async_copy(ones, acc.at[cidx.at[i]], sem, add=True)

            @pl.when(i >= WIN)
            def _roll():
                pltpu.make_async_copy(ones, acc.at[cidx.at[i]], sem).wait()

        @pl.loop(0, WIN)
        def _drain(i):
            pltpu.make_async_copy(ones, acc.at[cidx.at[0]], sem).wait()

    @pl.when(cid == 0)
    def _sc0():
        _deg_loop(Q0, sid * Q0)

    @pl.when(cid == 1)
    def _sc1():
        _deg_loop(Q1, NS * Q0 + sid * Q1)

    @pl.when(jnp.logical_and(cid == 1, sid >= NS - (NCHUNKS - TAIL_BASE)))
    def _tail():
        tc = TAIL_BASE + sid - (NS - (NCHUNKS - TAIL_BASE))
        pltpu.sync_copy(col_hbm.at[tc], tcol)
        pltpu.sync_copy(ones, acc.at[tcol], add=True)

    plsc.subcore_barrier()
    _copy_out(acc, out_hbm, cid, sid)


def _tc_mm_body(x_ref, w0_ref, w1_ref, y0_ref, y1_ref):
    x = x_ref[...]
    y0_ref[...] = jnp.dot(x, w0_ref[...], preferred_element_type=jnp.float32)
    y1_ref[...] = jnp.dot(x, w1_ref[...], preferred_element_type=jnp.float32)


def _tc_scale_body(y1_ref, d_ref, y1s_ref, dis_ref):
    deg = d_ref[0, :N_NODES, :] + d_ref[1, :N_NODES, :]
    dis = jnp.where(deg > 0.0, lax.rsqrt(deg), 0.0)
    dis_ref[...] = dis
    y1s_ref[pl.ds(0, N_NODES), :] = dis * y1_ref[...]
    y1s_ref[pl.ds(N_NODES, NP - N_NODES), :] = jnp.zeros(
        (NP - N_NODES, D_HID), jnp.float32)


def _tc2_body(y0_ref, a_ref, dis_ref, b1_ref, w0_ref, w1_ref,
              z0_ref, z1s_ref):
    dis = dis_ref[...]
    agg = a_ref[0, :N_NODES, :] + a_ref[1, :N_NODES, :]
    h = y0_ref[...] + dis * agg + b1_ref[...]
    h = jnp.maximum(h, 0.0)
    z0_ref[...] = jnp.dot(h, w0_ref[...], preferred_element_type=jnp.float32)
    z1 = jnp.dot(h, w1_ref[...], preferred_element_type=jnp.float32)
    z1s_ref[pl.ds(0, N_NODES), :] = dis * z1
    z1s_ref[pl.ds(N_NODES, NP - N_NODES), :] = jnp.zeros(
        (NP - N_NODES, D_HID), jnp.float32)


def _tc3_body(z0_ref, a_ref, dis_ref, b2_ref, out_ref):
    agg = a_ref[0, :N_NODES, :] + a_ref[1, :N_NODES, :]
    o = z0_ref[...] + dis_ref[...] * agg + b2_ref[...]
    m = jnp.max(o, axis=1, keepdims=True)
    s = jnp.sum(jnp.exp(o - m), axis=1, keepdims=True)
    out_ref[...] = o - m - jnp.log(s)


def kernel(x, edge_index, W1_0, W1_1, b1, W2_0, W2_1, b2):
    n = x.shape[0]
    row3 = edge_index[0].astype(jnp.int32).reshape(NCHUNKS, CHUNK)
    col3 = edge_index[1].astype(jnp.int32).reshape(NCHUNKS, CHUNK)

    degp = _sc_degree(col3)

    y0, y1 = pl.pallas_call(
        _tc_mm_body,
        out_shape=(
            jax.ShapeDtypeStruct((n, D_HID), jnp.float32),
            jax.ShapeDtypeStruct((n, D_HID), jnp.float32),
        ),
    )(x, W1_0, W1_1)

    y1s_p, dis = pl.pallas_call(
        _tc_scale_body,
        out_shape=(
            jax.ShapeDtypeStruct((NP, D_HID), jnp.float32),
            jax.ShapeDtypeStruct((n, D_HID), jnp.float32),
        ),
    )(y1, degp)

    agg1 = _sc_scatter(y1s_p, row3, col3)

    z0, z1s_p = pl.pallas_call(
        _tc2_body,
        out_shape=(
            jax.ShapeDtypeStruct((n, D_HID), jnp.float32),
            jax.ShapeDtypeStruct((NP, D_HID), jnp.float32),
        ),
    )(y0, agg1, dis, b1.reshape(1, D_HID), W2_0, W2_1)

    agg2 = _sc_scatter(z1s_p, row3, col3)

    out = pl.pallas_call(
        _tc3_body,
        out_shape=jax.ShapeDtypeStruct((n, D_HID), jnp.float32),
    )(z0, agg2, dis, b2.reshape(1, D_HID))
    return out


# speedup vs baseline: 1.3013x; 1.0121x over previous
import functools

import jax
import jax.numpy as jnp
from jax import lax
from jax.experimental import pallas as pl
from jax.experimental.pallas import tpu as pltpu
from jax.experimental.pallas import tpu_sc as plsc

N_NODES = 10000
N_EDGES = 320000
D_FEAT = 128
D_HID = 16

NC = 2
NS = 16
NW = NC * NS
L = 16

NP = 10240
RPT = NP // NS
CHUNK = 128
NCHUNKS = N_EDGES // CHUNK
Q0 = 96
Q1 = 60
TAIL_BASE = NS * (Q0 + Q1)
QMAX = max(Q0, Q1)
NB = 4

_mesh = plsc.VectorSubcoreMesh(core_axis_name="c", subcore_axis_name="s")


def _zero_fill(zbuf, acc, sid):
    for r in range(CHUNK):
        zbuf[r, :] = jnp.zeros((L,), jnp.float32)

    def body(j, _):
        pltpu.sync_copy(zbuf, acc.at[pl.ds(sid * RPT + j * CHUNK, CHUNK)])
        return 0

    lax.fori_loop(0, RPT // CHUNK, body, 0)


def _copy_out(acc, out_hbm, cid, sid):
    pltpu.sync_copy(acc.at[pl.ds(sid * RPT, RPT)],
                    out_hbm.at[cid, pl.ds(sid * RPT, RPT)])


@functools.partial(
    pl.kernel,
    out_type=jax.ShapeDtypeStruct((NC, NP, L), jnp.float32),
    mesh=_mesh,
    compiler_params=pltpu.CompilerParams(use_tc_tiling_on_sc=False),
    scratch_types=[
        pltpu.VMEM((QMAX, CHUNK), jnp.int32),
        pltpu.VMEM((QMAX, CHUNK), jnp.int32),
        [pltpu.VMEM((CHUNK, L), jnp.float32)] * NB,
        pltpu.VMEM((CHUNK,), jnp.int32),
        pltpu.VMEM((CHUNK,), jnp.int32),
        pltpu.VMEM_SHARED((NP, L), jnp.float32),
        [pltpu.SemaphoreType.DMA] * NB,
        [pltpu.SemaphoreType.DMA] * NB,
    ],
)
def _sc_scatter(y_hbm, row_hbm, col_hbm, out_hbm,
                ridx, cidx, rows, trow, tcol, acc, gsems, ssems):
    cid = lax.axis_index("c")
    sid = lax.axis_index("s")
    _zero_fill(rows[0], acc, sid)

    def _edge_loop(q, start):
        pltpu.sync_copy(row_hbm.at[pl.ds(start, q)], ridx.at[pl.ds(0, q)])
        pltpu.sync_copy(col_hbm.at[pl.ds(start, q)], cidx.at[pl.ds(0, q)])
        plsc.subcore_barrier()

        for j in range(NB - 1):
            pltpu.async_copy(y_hbm.at[ridx.at[j]], rows[j], gsems[j])

        @pl.loop(0, q // NB)
        def _steps(g):
            i0 = g * NB
            for b in range(NB):
                i = i0 + b
                pltpu.make_async_copy(y_hbm.at[ridx.at[i]], rows[b],
                                      gsems[b]).wait()

                pb = (b - 1) % NB
                @pl.when(i > 0)
                def _drain_prev():
                    pltpu.make_async_copy(rows[pb], acc.at[cidx.at[i]],
                                          ssems[pb]).wait()

                @pl.when(i + NB - 1 < q)
                def _prefetch():
                    pltpu.async_copy(y_hbm.at[ridx.at[i + NB - 1]], rows[pb],
                                     gsems[pb])

                pltpu.async_copy(rows[b], acc.at[cidx.at[i]], ssems[b],
                                 add=True)

        pltpu.make_async_copy(rows[NB - 1], acc.at[cidx.at[q - 1]],
                              ssems[NB - 1]).wait()

    @pl.when(cid == 0)
    def _sc0():
        _edge_loop(Q0, sid * Q0)

    @pl.when(cid == 1)
    def _sc1():
        _edge_loop(Q1, NS * Q0 + sid * Q1)

    @pl.when(jnp.logical_and(cid == 1, sid >= NS - (NCHUNKS - TAIL_BASE)))
    def _tail():
        tc = TAIL_BASE + sid - (NS - (NCHUNKS - TAIL_BASE))
        pltpu.sync_copy(row_hbm.at[tc], trow)
        pltpu.sync_copy(col_hbm.at[tc], tcol)
        pltpu.async_copy(y_hbm.at[trow], rows[0], gsems[0]).wait()
        pltpu.sync_copy(rows[0], acc.at[tcol], add=True)

    plsc.subcore_barrier()
    _copy_out(acc, out_hbm, cid, sid)


@functools.partial(
    pl.kernel,
    out_type=jax.ShapeDtypeStruct((NC, NP, L), jnp.float32),
    mesh=_mesh,
    compiler_params=pltpu.CompilerParams(use_tc_tiling_on_sc=False),
    scratch_types=[
        pltpu.VMEM((QMAX, CHUNK), jnp.int32),
        pltpu.VMEM((CHUNK, L), jnp.float32),
        pltpu.VMEM((CHUNK,), jnp.int32),
        pltpu.VMEM_SHARED((NP, L), jnp.float32),
        pltpu.SemaphoreType.DMA,
    ],
)
def _sc_degree(col_hbm, out_hbm, cidx, ones, tcol, acc, sem):
    cid = lax.axis_index("c")
    sid = lax.axis_index("s")
    _zero_fill(ones, acc, sid)
    for r in range(CHUNK):
        ones[r, :] = jnp.ones((L,), jnp.float32)

    WIN = 8

    def _deg_loop(q, start):
        pltpu.sync_copy(col_hbm.at[pl.ds(start, q)], cidx.at[pl.ds(0, q)])
        plsc.subcore_barrier()

        @pl.loop(0, q)
        def _steps(i):
            pltpu.async_copy(ones, acc.at[cidx.at[i]], sem, add=True)

            @pl.when(i >= WIN)
            def _roll():
                pltpu.make_async_copy(ones, acc.at[cidx.at[i]], sem).wait()

        @pl.loop(0, WIN)
        def _drain(i):
            pltpu.make_async_copy(ones, acc.at[cidx.at[0]], sem).wait()

    @pl.when(cid == 0)
    def _sc0():
        _deg_loop(Q0, sid * Q0)

    @pl.when(cid == 1)
    def _sc1():
        _deg_loop(Q1, NS * Q0 + sid * Q1)

    @pl.when(jnp.logical_and(cid == 1, sid >= NS - (NCHUNKS - TAIL_BASE)))
    def _tail():
        tc = TAIL_BASE + sid - (NS - (NCHUNKS - TAIL_BASE))
        pltpu.sync_copy(col_hbm.at[tc], tcol)
        pltpu.sync_copy(ones, acc.at[tcol], add=True)

    plsc.subcore_barrier()
    _copy_out(acc, out_hbm, cid, sid)


def _tc_mm_body(x_ref, w0_ref, w1_ref, y0_ref, y1_ref):
    x = x_ref[...]
    y0_ref[...] = jnp.dot(x, w0_ref[...], preferred_element_type=jnp.float32)
    y1_ref[...] = jnp.dot(x, w1_ref[...], preferred_element_type=jnp.float32)


def _tc_scale_body(y1_ref, d_ref, y1s_ref, dis_ref):
    deg = d_ref[0, :N_NODES, :] + d_ref[1, :N_NODES, :]
    dis = jnp.where(deg > 0.0, lax.rsqrt(deg), 0.0)
    dis_ref[...] = dis
    y1s_ref[pl.ds(0, N_NODES), :] = dis * y1_ref[...]
    y1s_ref[pl.ds(N_NODES, NP - N_NODES), :] = jnp.zeros(
        (NP - N_NODES, D_HID), jnp.float32)


def _tc2_body(y0_ref, a_ref, dis_ref, b1_ref, w0_ref, w1_ref,
              z0_ref, z1s_ref):
    dis = dis_ref[...]
    agg = a_ref[0, :N_NODES, :] + a_ref[1, :N_NODES, :]
    h = y0_ref[...] + dis * agg + b1_ref[...]
    h = jnp.maximum(h, 0.0)
    z0_ref[...] = jnp.dot(h, w0_ref[...], preferred_element_type=jnp.float32)
    z1 = jnp.dot(h, w1_ref[...], preferred_element_type=jnp.float32)
    z1s_ref[pl.ds(0, N_NODES), :] = dis * z1
    z1s_ref[pl.ds(N_NODES, NP - N_NODES), :] = jnp.zeros(
        (NP - N_NODES, D_HID), jnp.float32)


def _tc3_body(z0_ref, a_ref, dis_ref, b2_ref, out_ref):
    agg = a_ref[0, :N_NODES, :] + a_ref[1, :N_NODES, :]
    o = z0_ref[...] + dis_ref[...] * agg + b2_ref[...]
    m = jnp.max(o, axis=1, keepdims=True)
    s = jnp.sum(jnp.exp(o - m), axis=1, keepdims=True)
    out_ref[...] = o - m - jnp.log(s)


def kernel(x, edge_index, W1_0, W1_1, b1, W2_0, W2_1, b2):
    n = x.shape[0]
    row3 = edge_index[0].astype(jnp.int32).reshape(NCHUNKS, CHUNK)
    col3 = edge_index[1].astype(jnp.int32).reshape(NCHUNKS, CHUNK)

    degp = _sc_degree(col3)

    y0, y1 = pl.pallas_call(
        _tc_mm_body,
        out_shape=(
            jax.ShapeDtypeStruct((n, D_HID), jnp.float32),
            jax.ShapeDtypeStruct((n, D_HID), jnp.float32),
        ),
    )(x, W1_0, W1_1)

    y1s_p, dis = pl.pallas_call(
        _tc_scale_body,
        out_shape=(
            jax.ShapeDtypeStruct((NP, D_HID), jnp.float32),
            jax.ShapeDtypeStruct((n, D_HID), jnp.float32),
        ),
    )(y1, degp)

    agg1 = _sc_scatter(y1s_p, row3, col3)

    z0, z1s_p = pl.pallas_call(
        _tc2_body,
        out_shape=(
            jax.ShapeDtypeStruct((n, D_HID), jnp.float32),
            jax.ShapeDtypeStruct((NP, D_HID), jnp.float32),
        ),
    )(y0, agg1, dis, b1.reshape(1, D_HID), W2_0, W2_1)

    agg2 = _sc_scatter(z1s_p, row3, col3)

    out = pl.pallas_call(
        _tc3_body,
        out_shape=jax.ShapeDtypeStruct((n, D_HID), jnp.float32),
    )(z0, agg2, dis, b2.reshape(1, D_HID))
    return out


# speedup vs baseline: 1.3206x; 1.0149x over previous
import functools

import jax
import jax.numpy as jnp
from jax import lax
from jax.experimental import pallas as pl
from jax.experimental.pallas import tpu as pltpu
from jax.experimental.pallas import tpu_sc as plsc

N_NODES = 10000
N_EDGES = 320000
D_FEAT = 128
D_HID = 16

NC = 2
NS = 16
NW = NC * NS
L = 16

NP = 10240
RPT = NP // NS
CHUNK = 128
NCHUNKS = N_EDGES // CHUNK
Q0 = 92
Q1 = 64
TAIL_BASE = NS * (Q0 + Q1)
QMAX = max(Q0, Q1)
NB = 4

_mesh = plsc.VectorSubcoreMesh(core_axis_name="c", subcore_axis_name="s")


def _zero_fill(zbuf, acc, sid):
    for r in range(CHUNK):
        zbuf[r, :] = jnp.zeros((L,), jnp.float32)

    def body(j, _):
        pltpu.sync_copy(zbuf, acc.at[pl.ds(sid * RPT + j * CHUNK, CHUNK)])
        return 0

    lax.fori_loop(0, RPT // CHUNK, body, 0)


def _copy_out(acc, out_hbm, cid, sid):
    pltpu.sync_copy(acc.at[pl.ds(sid * RPT, RPT)],
                    out_hbm.at[cid, pl.ds(sid * RPT, RPT)])


@functools.partial(
    pl.kernel,
    out_type=jax.ShapeDtypeStruct((NC, NP, L), jnp.float32),
    mesh=_mesh,
    compiler_params=pltpu.CompilerParams(use_tc_tiling_on_sc=False),
    scratch_types=[
        pltpu.VMEM((QMAX, CHUNK), jnp.int32),
        pltpu.VMEM((QMAX, CHUNK), jnp.int32),
        [pltpu.VMEM((CHUNK, L), jnp.float32)] * NB,
        pltpu.VMEM((CHUNK,), jnp.int32),
        pltpu.VMEM((CHUNK,), jnp.int32),
        pltpu.VMEM_SHARED((NP, L), jnp.float32),
        [pltpu.SemaphoreType.DMA] * NB,
        [pltpu.SemaphoreType.DMA] * NB,
    ],
)
def _sc_scatter(y_hbm, row_hbm, col_hbm, out_hbm,
                ridx, cidx, rows, trow, tcol, acc, gsems, ssems):
    cid = lax.axis_index("c")
    sid = lax.axis_index("s")
    _zero_fill(rows[0], acc, sid)

    def _edge_loop(q, start):
        pltpu.sync_copy(row_hbm.at[pl.ds(start, q)], ridx.at[pl.ds(0, q)])
        pltpu.sync_copy(col_hbm.at[pl.ds(start, q)], cidx.at[pl.ds(0, q)])
        plsc.subcore_barrier()

        for j in range(NB - 1):
            pltpu.async_copy(y_hbm.at[ridx.at[j]], rows[j], gsems[j])

        @pl.loop(0, q // NB)
        def _steps(g):
            i0 = g * NB
            for b in range(NB):
                i = i0 + b
                pltpu.make_async_copy(y_hbm.at[ridx.at[i]], rows[b],
                                      gsems[b]).wait()

                pb = (b - 1) % NB
                @pl.when(i > 0)
                def _drain_prev():
                    pltpu.make_async_copy(rows[pb], acc.at[cidx.at[i]],
                                          ssems[pb]).wait()

                @pl.when(i + NB - 1 < q)
                def _prefetch():
                    pltpu.async_copy(y_hbm.at[ridx.at[i + NB - 1]], rows[pb],
                                     gsems[pb])

                pltpu.async_copy(rows[b], acc.at[cidx.at[i]], ssems[b],
                                 add=True)

        pltpu.make_async_copy(rows[NB - 1], acc.at[cidx.at[q - 1]],
                              ssems[NB - 1]).wait()

    @pl.when(cid == 0)
    def _sc0():
        _edge_loop(Q0, sid * Q0)

    @pl.when(cid == 1)
    def _sc1():
        _edge_loop(Q1, NS * Q0 + sid * Q1)

    @pl.when(jnp.logical_and(cid == 1, sid >= NS - (NCHUNKS - TAIL_BASE)))
    def _tail():
        tc = TAIL_BASE + sid - (NS - (NCHUNKS - TAIL_BASE))
        pltpu.sync_copy(row_hbm.at[tc], trow)
        pltpu.sync_copy(col_hbm.at[tc], tcol)
        pltpu.async_copy(y_hbm.at[trow], rows[0], gsems[0]).wait()
        pltpu.sync_copy(rows[0], acc.at[tcol], add=True)

    plsc.subcore_barrier()
    _copy_out(acc, out_hbm, cid, sid)


@functools.partial(
    pl.kernel,
    out_type=jax.ShapeDtypeStruct((NC, NP, L), jnp.float32),
    mesh=_mesh,
    compiler_params=pltpu.CompilerParams(use_tc_tiling_on_sc=False),
    scratch_types=[
        pltpu.VMEM((QMAX, CHUNK), jnp.int32),
        pltpu.VMEM((CHUNK, L), jnp.float32),
        pltpu.VMEM((CHUNK,), jnp.int32),
        pltpu.VMEM_SHARED((NP, L), jnp.float32),
        pltpu.SemaphoreType.DMA,
    ],
)
def _sc_degree(col_hbm, out_hbm, cidx, ones, tcol, acc, sem):
    cid = lax.axis_index("c")
    sid = lax.axis_index("s")
    _zero_fill(ones, acc, sid)
    for r in range(CHUNK):
        ones[r, :] = jnp.ones((L,), jnp.float32)

    WIN = 8

    def _deg_loop(q, start):
        pltpu.sync_copy(col_hbm.at[pl.ds(start, q)], cidx.at[pl.ds(0, q)])
        plsc.subcore_barrier()

        @pl.loop(0, q)
        def _steps(i):
            pltpu.async_copy(ones, acc.at[cidx.at[i]], sem, add=True)

            @pl.when(i >= WIN)
            def _roll():
                pltpu.make_async_copy(ones, acc.at[cidx.at[i]], sem).wait()

        @pl.loop(0, WIN)
        def _drain(i):
            pltpu.make_async_copy(ones, acc.at[cidx.at[0]], sem).wait()

    @pl.when(cid == 0)
    def _sc0():
        _deg_loop(Q0, sid * Q0)

    @pl.when(cid == 1)
    def _sc1():
        _deg_loop(Q1, NS * Q0 + sid * Q1)

    @pl.when(jnp.logical_and(cid == 1, sid >= NS - (NCHUNKS - TAIL_BASE)))
    def _tail():
        tc = TAIL_BASE + sid - (NS - (NCHUNKS - TAIL_BASE))
        pltpu.sync_copy(col_hbm.at[tc], tcol)
        pltpu.sync_copy(ones, acc.at[tcol], add=True)

    plsc.subcore_barrier()
    _copy_out(acc, out_hbm, cid, sid)


def _tc_mm_body(x_ref, w0_ref, w1_ref, y0_ref, y1_ref):
    x = x_ref[...]
    y0_ref[...] = jnp.dot(x, w0_ref[...], preferred_element_type=jnp.float32)
    y1_ref[...] = jnp.dot(x, w1_ref[...], preferred_element_type=jnp.float32)


def _tc_scale_body(y1_ref, d_ref, y1s_ref, dis_ref):
    deg = d_ref[0, :N_NODES, :] + d_ref[1, :N_NODES, :]
    dis = jnp.where(deg > 0.0, lax.rsqrt(deg), 0.0)
    dis_ref[...] = dis
    y1s_ref[pl.ds(0, N_NODES), :] = dis * y1_ref[...]
    y1s_ref[pl.ds(N_NODES, NP - N_NODES), :] = jnp.zeros(
        (NP - N_NODES, D_HID), jnp.float32)


def _tc2_body(y0_ref, a_ref, dis_ref, b1_ref, w0_ref, w1_ref,
              z0_ref, z1s_ref):
    dis = dis_ref[...]
    agg = a_ref[0, :N_NODES, :] + a_ref[1, :N_NODES, :]
    h = y0_ref[...] + dis * agg + b1_ref[...]
    h = jnp.maximum(h, 0.0)
    z0_ref[...] = jnp.dot(h, w0_ref[...], preferred_element_type=jnp.float32)
    z1 = jnp.dot(h, w1_ref[...], preferred_element_type=jnp.float32)
    z1s_ref[pl.ds(0, N_NODES), :] = dis * z1
    z1s_ref[pl.ds(N_NODES, NP - N_NODES), :] = jnp.zeros(
        (NP - N_NODES, D_HID), jnp.float32)


def _tc3_body(z0_ref, a_ref, dis_ref, b2_ref, out_ref):
    agg = a_ref[0, :N_NODES, :] + a_ref[1, :N_NODES, :]
    o = z0_ref[...] + dis_ref[...] * agg + b2_ref[...]
    m = jnp.max(o, axis=1, keepdims=True)
    s = jnp.sum(jnp.exp(o - m), axis=1, keepdims=True)
    out_ref[...] = o - m - jnp.log(s)


def kernel(x, edge_index, W1_0, W1_1, b1, W2_0, W2_1, b2):
    n = x.shape[0]
    row3 = edge_index[0].astype(jnp.int32).reshape(NCHUNKS, CHUNK)
    col3 = edge_index[1].astype(jnp.int32).reshape(NCHUNKS, CHUNK)

    degp = _sc_degree(col3)

    y0, y1 = pl.pallas_call(
        _tc_mm_body,
        out_shape=(
            jax.ShapeDtypeStruct((n, D_HID), jnp.float32),
            jax.ShapeDtypeStruct((n, D_HID), jnp.float32),
        ),
    )(x, W1_0, W1_1)

    y1s_p, dis = pl.pallas_call(
        _tc_scale_body,
        out_shape=(
            jax.ShapeDtypeStruct((NP, D_HID), jnp.float32),
            jax.ShapeDtypeStruct((n, D_HID), jnp.float32),
        ),
    )(y1, degp)

    agg1 = _sc_scatter(y1s_p, row3, col3)

    z0, z1s_p = pl.pallas_call(
        _tc2_body,
        out_shape=(
            jax.ShapeDtypeStruct((n, D_HID), jnp.float32),
            jax.ShapeDtypeStruct((NP, D_HID), jnp.float32),
        ),
    )(y0, agg1, dis, b1.reshape(1, D_HID), W2_0, W2_1)

    agg2 = _sc_scatter(z1s_p, row3, col3)

    out = pl.pallas_call(
        _tc3_body,
        out_shape=jax.ShapeDtypeStruct((n, D_HID), jnp.float32),
    )(z0, agg2, dis, b2.reshape(1, D_HID))
    return out


# speedup vs baseline: 1.3534x; 1.0248x over previous
import functools

import jax
import jax.numpy as jnp
from jax import lax
from jax.experimental import pallas as pl
from jax.experimental.pallas import tpu as pltpu
from jax.experimental.pallas import tpu_sc as plsc

N_NODES = 10000
N_EDGES = 320000
D_FEAT = 128
D_HID = 16

NC = 2
NS = 16
NW = NC * NS
L = 16

NP = 10240
RPT = NP // NS
CHUNK = 128
NCHUNKS = N_EDGES // CHUNK
Q0 = 84
Q1 = 72
TAIL_BASE = NS * (Q0 + Q1)
QMAX = max(Q0, Q1)
NB = 4

_mesh = plsc.VectorSubcoreMesh(core_axis_name="c", subcore_axis_name="s")


def _zero_fill(zbuf, acc, sid):
    for r in range(CHUNK):
        zbuf[r, :] = jnp.zeros((L,), jnp.float32)

    def body(j, _):
        pltpu.sync_copy(zbuf, acc.at[pl.ds(sid * RPT + j * CHUNK, CHUNK)])
        return 0

    lax.fori_loop(0, RPT // CHUNK, body, 0)


def _copy_out(acc, out_hbm, cid, sid):
    pltpu.sync_copy(acc.at[pl.ds(sid * RPT, RPT)],
                    out_hbm.at[cid, pl.ds(sid * RPT, RPT)])


@functools.partial(
    pl.kernel,
    out_type=jax.ShapeDtypeStruct((NC, NP, L), jnp.float32),
    mesh=_mesh,
    compiler_params=pltpu.CompilerParams(use_tc_tiling_on_sc=False),
    scratch_types=[
        pltpu.VMEM((QMAX, CHUNK), jnp.int32),
        pltpu.VMEM((QMAX, CHUNK), jnp.int32),
        [pltpu.VMEM((CHUNK, L), jnp.float32)] * NB,
        pltpu.VMEM((CHUNK,), jnp.int32),
        pltpu.VMEM((CHUNK,), jnp.int32),
        pltpu.VMEM_SHARED((NP, L), jnp.float32),
        [pltpu.SemaphoreType.DMA] * NB,
        [pltpu.SemaphoreType.DMA] * NB,
    ],
)
def _sc_scatter(y_hbm, row_hbm, col_hbm, out_hbm,
                ridx, cidx, rows, trow, tcol, acc, gsems, ssems):
    cid = lax.axis_index("c")
    sid = lax.axis_index("s")
    _zero_fill(rows[0], acc, sid)

    def _edge_loop(q, start):
        pltpu.sync_copy(row_hbm.at[pl.ds(start, q)], ridx.at[pl.ds(0, q)])
        pltpu.sync_copy(col_hbm.at[pl.ds(start, q)], cidx.at[pl.ds(0, q)])
        plsc.subcore_barrier()

        for j in range(NB - 1):
            pltpu.async_copy(y_hbm.at[ridx.at[j]], rows[j], gsems[j])

        @pl.loop(0, q // NB)
        def _steps(g):
            i0 = g * NB
            for b in range(NB):
                i = i0 + b
                pltpu.make_async_copy(y_hbm.at[ridx.at[i]], rows[b],
                                      gsems[b]).wait()

                pb = (b - 1) % NB
                @pl.when(i > 0)
                def _drain_prev():
                    pltpu.make_async_copy(rows[pb], acc.at[cidx.at[i]],
                                          ssems[pb]).wait()

                @pl.when(i + NB - 1 < q)
                def _prefetch():
                    pltpu.async_copy(y_hbm.at[ridx.at[i + NB - 1]], rows[pb],
                                     gsems[pb])

                pltpu.async_copy(rows[b], acc.at[cidx.at[i]], ssems[b],
                                 add=True)

        pltpu.make_async_copy(rows[NB - 1], acc.at[cidx.at[q - 1]],
                              ssems[NB - 1]).wait()

    @pl.when(cid == 0)
    def _sc0():
        _edge_loop(Q0, sid * Q0)

    @pl.when(cid == 1)
    def _sc1():
        _edge_loop(Q1, NS * Q0 + sid * Q1)

    @pl.when(jnp.logical_and(cid == 1, sid >= NS - (NCHUNKS - TAIL_BASE)))
    def _tail():
        tc = TAIL_BASE + sid - (NS - (NCHUNKS - TAIL_BASE))
        pltpu.sync_copy(row_hbm.at[tc], trow)
        pltpu.sync_copy(col_hbm.at[tc], tcol)
        pltpu.async_copy(y_hbm.at[trow], rows[0], gsems[0]).wait()
        pltpu.sync_copy(rows[0], acc.at[tcol], add=True)

    plsc.subcore_barrier()
    _copy_out(acc, out_hbm, cid, sid)


@functools.partial(
    pl.kernel,
    out_type=jax.ShapeDtypeStruct((NC, NP, L), jnp.float32),
    mesh=_mesh,
    compiler_params=pltpu.CompilerParams(use_tc_tiling_on_sc=False),
    scratch_types=[
        pltpu.VMEM((QMAX, CHUNK), jnp.int32),
        pltpu.VMEM((CHUNK, L), jnp.float32),
        pltpu.VMEM((CHUNK,), jnp.int32),
        pltpu.VMEM_SHARED((NP, L), jnp.float32),
        pltpu.SemaphoreType.DMA,
    ],
)
def _sc_degree(col_hbm, out_hbm, cidx, ones, tcol, acc, sem):
    cid = lax.axis_index("c")
    sid = lax.axis_index("s")
    _zero_fill(ones, acc, sid)
    for r in range(CHUNK):
        ones[r, :] = jnp.ones((L,), jnp.float32)

    WIN = 8

    def _deg_loop(q, start):
        pltpu.sync_copy(col_hbm.at[pl.ds(start, q)], cidx.at[pl.ds(0, q)])
        plsc.subcore_barrier()

        @pl.loop(0, q)
        def _steps(i):
            pltpu.async_copy(ones, acc.at[cidx.at[i]], sem, add=True)

            @pl.when(i >= WIN)
            def _roll():
                pltpu.make_async_copy(ones, acc.at[cidx.at[i]], sem).wait()

        @pl.loop(0, WIN)
        def _drain(i):
            pltpu.make_async_copy(ones, acc.at[cidx.at[0]], sem).wait()

    @pl.when(cid == 0)
    def _sc0():
        _deg_loop(Q0, sid * Q0)

    @pl.when(cid == 1)
    def _sc1():
        _deg_loop(Q1, NS * Q0 + sid * Q1)

    @pl.when(jnp.logical_and(cid == 1, sid >= NS - (NCHUNKS - TAIL_BASE)))
    def _tail():
        tc = TAIL_BASE + sid - (NS - (NCHUNKS - TAIL_BASE))
        pltpu.sync_copy(col_hbm.at[tc], tcol)
        pltpu.sync_copy(ones, acc.at[tcol], add=True)

    plsc.subcore_barrier()
    _copy_out(acc, out_hbm, cid, sid)


def _tc_mm_body(x_ref, w0_ref, w1_ref, y0_ref, y1_ref):
    x = x_ref[...]
    y0_ref[...] = jnp.dot(x, w0_ref[...], preferred_element_type=jnp.float32)
    y1_ref[...] = jnp.dot(x, w1_ref[...], preferred_element_type=jnp.float32)


def _tc_scale_body(y1_ref, d_ref, y1s_ref, dis_ref):
    deg = d_ref[0, :N_NODES, :] + d_ref[1, :N_NODES, :]
    dis = jnp.where(deg > 0.0, lax.rsqrt(deg), 0.0)
    dis_ref[...] = dis
    y1s_ref[pl.ds(0, N_NODES), :] = dis * y1_ref[...]
    y1s_ref[pl.ds(N_NODES, NP - N_NODES), :] = jnp.zeros(
        (NP - N_NODES, D_HID), jnp.float32)


def _tc2_body(y0_ref, a_ref, dis_ref, b1_ref, w0_ref, w1_ref,
              z0_ref, z1s_ref):
    dis = dis_ref[...]
    agg = a_ref[0, :N_NODES, :] + a_ref[1, :N_NODES, :]
    h = y0_ref[...] + dis * agg + b1_ref[...]
    h = jnp.maximum(h, 0.0)
    z0_ref[...] = jnp.dot(h, w0_ref[...], preferred_element_type=jnp.float32)
    z1 = jnp.dot(h, w1_ref[...], preferred_element_type=jnp.float32)
    z1s_ref[pl.ds(0, N_NODES), :] = dis * z1
    z1s_ref[pl.ds(N_NODES, NP - N_NODES), :] = jnp.zeros(
        (NP - N_NODES, D_HID), jnp.float32)


def _tc3_body(z0_ref, a_ref, dis_ref, b2_ref, out_ref):
    agg = a_ref[0, :N_NODES, :] + a_ref[1, :N_NODES, :]
    o = z0_ref[...] + dis_ref[...] * agg + b2_ref[...]
    m = jnp.max(o, axis=1, keepdims=True)
    s = jnp.sum(jnp.exp(o - m), axis=1, keepdims=True)
    out_ref[...] = o - m - jnp.log(s)


def kernel(x, edge_index, W1_0, W1_1, b1, W2_0, W2_1, b2):
    n = x.shape[0]
    row3 = edge_index[0].astype(jnp.int32).reshape(NCHUNKS, CHUNK)
    col3 = edge_index[1].astype(jnp.int32).reshape(NCHUNKS, CHUNK)

    degp = _sc_degree(col3)

    y0, y1 = pl.pallas_call(
        _tc_mm_body,
        out_shape=(
            jax.ShapeDtypeStruct((n, D_HID), jnp.float32),
            jax.ShapeDtypeStruct((n, D_HID), jnp.float32),
        ),
    )(x, W1_0, W1_1)

    y1s_p, dis = pl.pallas_call(
        _tc_scale_body,
        out_shape=(
            jax.ShapeDtypeStruct((NP, D_HID), jnp.float32),
            jax.ShapeDtypeStruct((n, D_HID), jnp.float32),
        ),
    )(y1, degp)

    agg1 = _sc_scatter(y1s_p, row3, col3)

    z0, z1s_p = pl.pallas_call(
        _tc2_body,
        out_shape=(
            jax.ShapeDtypeStruct((n, D_HID), jnp.float32),
            jax.ShapeDtypeStruct((NP, D_HID), jnp.float32),
        ),
    )(y0, agg1, dis, b1.reshape(1, D_HID), W2_0, W2_1)

    agg2 = _sc_scatter(z1s_p, row3, col3)

    out = pl.pallas_call(
        _tc3_body,
        out_shape=jax.ShapeDtypeStruct((n, D_HID), jnp.float32),
    )(z0, agg2, dis, b2.reshape(1, D_HID))
    return out


# speedup vs baseline: 1.3723x; 1.0140x over previous
import functools

import jax
import jax.numpy as jnp
from jax import lax
from jax.experimental import pallas as pl
from jax.experimental.pallas import tpu as pltpu
from jax.experimental.pallas import tpu_sc as plsc

N_NODES = 10000
N_EDGES = 320000
D_FEAT = 128
D_HID = 16

NC = 2
NS = 16
NW = NC * NS
L = 16

NP = 10240
RPT = NP // NS
CHUNK = 128
NCHUNKS = N_EDGES // CHUNK
Q0 = 80
Q1 = 76
TAIL_BASE = NS * (Q0 + Q1)
QMAX = max(Q0, Q1)
NB = 4
assert Q0 % NB == 0 and Q1 % NB == 0

_mesh = plsc.VectorSubcoreMesh(core_axis_name="c", subcore_axis_name="s")


def _zero_fill(zbuf, acc, sid):
    for r in range(CHUNK):
        zbuf[r, :] = jnp.zeros((L,), jnp.float32)

    def body(j, _):
        pltpu.sync_copy(zbuf, acc.at[pl.ds(sid * RPT + j * CHUNK, CHUNK)])
        return 0

    lax.fori_loop(0, RPT // CHUNK, body, 0)


def _copy_out(acc, out_hbm, cid, sid):
    pltpu.sync_copy(acc.at[pl.ds(sid * RPT, RPT)],
                    out_hbm.at[cid, pl.ds(sid * RPT, RPT)])


@functools.partial(
    pl.kernel,
    out_type=jax.ShapeDtypeStruct((NC, NP, L), jnp.float32),
    mesh=_mesh,
    compiler_params=pltpu.CompilerParams(use_tc_tiling_on_sc=False),
    scratch_types=[
        pltpu.VMEM((QMAX, CHUNK), jnp.int32),
        pltpu.VMEM((QMAX, CHUNK), jnp.int32),
        [pltpu.VMEM((CHUNK, L), jnp.float32)] * NB,
        pltpu.VMEM((CHUNK,), jnp.int32),
        pltpu.VMEM((CHUNK,), jnp.int32),
        pltpu.VMEM_SHARED((NP, L), jnp.float32),
        [pltpu.SemaphoreType.DMA] * NB,
        [pltpu.SemaphoreType.DMA] * NB,
    ],
)
def _sc_scatter(y_hbm, row_hbm, col_hbm, out_hbm,
                ridx, cidx, rows, trow, tcol, acc, gsems, ssems):
    cid = lax.axis_index("c")
    sid = lax.axis_index("s")
    _zero_fill(rows[0], acc, sid)

    def _edge_loop(q, start):
        pltpu.sync_copy(row_hbm.at[pl.ds(start, q)], ridx.at[pl.ds(0, q)])
        pltpu.sync_copy(col_hbm.at[pl.ds(start, q)], cidx.at[pl.ds(0, q)])
        plsc.subcore_barrier()

        for j in range(NB - 1):
            pltpu.async_copy(y_hbm.at[ridx.at[j]], rows[j], gsems[j])

        @pl.loop(0, q // NB)
        def _steps(g):
            i0 = g * NB
            for b in range(NB):
                i = i0 + b
                pltpu.make_async_copy(y_hbm.at[ridx.at[i]], rows[b],
                                      gsems[b]).wait()

                pb = (b - 1) % NB
                @pl.when(i > 0)
                def _drain_prev():
                    pltpu.make_async_copy(rows[pb], acc.at[cidx.at[i]],
                                          ssems[pb]).wait()

                @pl.when(i + NB - 1 < q)
                def _prefetch():
                    pltpu.async_copy(y_hbm.at[ridx.at[i + NB - 1]], rows[pb],
                                     gsems[pb])

                pltpu.async_copy(rows[b], acc.at[cidx.at[i]], ssems[b],
                                 add=True)

        pltpu.make_async_copy(rows[NB - 1], acc.at[cidx.at[q - 1]],
                              ssems[NB - 1]).wait()

    @pl.when(cid == 0)
    def _sc0():
        _edge_loop(Q0, sid * Q0)

    @pl.when(cid == 1)
    def _sc1():
        _edge_loop(Q1, NS * Q0 + sid * Q1)

    @pl.when(jnp.logical_and(cid == 1, sid >= NS - (NCHUNKS - TAIL_BASE)))
    def _tail():
        tc = TAIL_BASE + sid - (NS - (NCHUNKS - TAIL_BASE))
        pltpu.sync_copy(row_hbm.at[tc], trow)
        pltpu.sync_copy(col_hbm.at[tc], tcol)
        pltpu.async_copy(y_hbm.at[trow], rows[0], gsems[0]).wait()
        pltpu.sync_copy(rows[0], acc.at[tcol], add=True)

    plsc.subcore_barrier()
    _copy_out(acc, out_hbm, cid, sid)


@functools.partial(
    pl.kernel,
    out_type=jax.ShapeDtypeStruct((NC, NP, L), jnp.float32),
    mesh=_mesh,
    compiler_params=pltpu.CompilerParams(use_tc_tiling_on_sc=False),
    scratch_types=[
        pltpu.VMEM((QMAX, CHUNK), jnp.int32),
        pltpu.VMEM((CHUNK, L), jnp.float32),
        pltpu.VMEM((CHUNK,), jnp.int32),
        pltpu.VMEM_SHARED((NP, L), jnp.float32),
        pltpu.SemaphoreType.DMA,
    ],
)
def _sc_degree(col_hbm, out_hbm, cidx, ones, tcol, acc, sem):
    cid = lax.axis_index("c")
    sid = lax.axis_index("s")
    _zero_fill(ones, acc, sid)
    for r in range(CHUNK):
        ones[r, :] = jnp.ones((L,), jnp.float32)

    WIN = 8

    def _deg_loop(q, start):
        pltpu.sync_copy(col_hbm.at[pl.ds(start, q)], cidx.at[pl.ds(0, q)])
        plsc.subcore_barrier()

        @pl.loop(0, q)
        def _steps(i):
            pltpu.async_copy(ones, acc.at[cidx.at[i]], sem, add=True)

            @pl.when(i >= WIN)
            def _roll():
                pltpu.make_async_copy(ones, acc.at[cidx.at[i]], sem).wait()

        @pl.loop(0, WIN)
        def _drain(i):
            pltpu.make_async_copy(ones, acc.at[cidx.at[0]], sem).wait()

    @pl.when(cid == 0)
    def _sc0():
        _deg_loop(Q0, sid * Q0)

    @pl.when(cid == 1)
    def _sc1():
        _deg_loop(Q1, NS * Q0 + sid * Q1)

    @pl.when(jnp.logical_and(cid == 1, sid >= NS - (NCHUNKS - TAIL_BASE)))
    def _tail():
        tc = TAIL_BASE + sid - (NS - (NCHUNKS - TAIL_BASE))
        pltpu.sync_copy(col_hbm.at[tc], tcol)
        pltpu.sync_copy(ones, acc.at[tcol], add=True)

    plsc.subcore_barrier()
    _copy_out(acc, out_hbm, cid, sid)


def _tc_mm_body(x_ref, w0_ref, w1_ref, y0_ref, y1_ref):
    x = x_ref[...]
    y0_ref[...] = jnp.dot(x, w0_ref[...], preferred_element_type=jnp.float32)
    y1_ref[...] = jnp.dot(x, w1_ref[...], preferred_element_type=jnp.float32)


def _tc_scale_body(y1_ref, d_ref, y1s_ref, dis_ref):
    deg = d_ref[0, :N_NODES, :] + d_ref[1, :N_NODES, :]
    dis = jnp.where(deg > 0.0, lax.rsqrt(deg), 0.0)
    dis_ref[...] = dis
    y1s_ref[pl.ds(0, N_NODES), :] = dis * y1_ref[...]
    y1s_ref[pl.ds(N_NODES, NP - N_NODES), :] = jnp.zeros(
        (NP - N_NODES, D_HID), jnp.float32)


def _tc2_body(y0_ref, a_ref, dis_ref, b1_ref, w0_ref, w1_ref,
              z0_ref, z1s_ref):
    dis = dis_ref[...]
    agg = a_ref[0, :N_NODES, :] + a_ref[1, :N_NODES, :]
    h = y0_ref[...] + dis * agg + b1_ref[...]
    h = jnp.maximum(h, 0.0)
    z0_ref[...] = jnp.dot(h, w0_ref[...], preferred_element_type=jnp.float32)
    z1 = jnp.dot(h, w1_ref[...], preferred_element_type=jnp.float32)
    z1s_ref[pl.ds(0, N_NODES), :] = dis * z1
    z1s_ref[pl.ds(N_NODES, NP - N_NODES), :] = jnp.zeros(
        (NP - N_NODES, D_HID), jnp.float32)


def _tc3_body(z0_ref, a_ref, dis_ref, b2_ref, out_ref):
    agg = a_ref[0, :N_NODES, :] + a_ref[1, :N_NODES, :]
    o = z0_ref[...] + dis_ref[...] * agg + b2_ref[...]
    m = jnp.max(o, axis=1, keepdims=True)
    s = jnp.sum(jnp.exp(o - m), axis=1, keepdims=True)
    out_ref[...] = o - m - jnp.log(s)


def kernel(x, edge_index, W1_0, W1_1, b1, W2_0, W2_1, b2):
    n = x.shape[0]
    row3 = edge_index[0].astype(jnp.int32).reshape(NCHUNKS, CHUNK)
    col3 = edge_index[1].astype(jnp.int32).reshape(NCHUNKS, CHUNK)

    degp = _sc_degree(col3)

    y0, y1 = pl.pallas_call(
        _tc_mm_body,
        out_shape=(
            jax.ShapeDtypeStruct((n, D_HID), jnp.float32),
            jax.ShapeDtypeStruct((n, D_HID), jnp.float32),
        ),
    )(x, W1_0, W1_1)

    y1s_p, dis = pl.pallas_call(
        _tc_scale_body,
        out_shape=(
            jax.ShapeDtypeStruct((NP, D_HID), jnp.float32),
            jax.ShapeDtypeStruct((n, D_HID), jnp.float32),
        ),
    )(y1, degp)

    agg1 = _sc_scatter(y1s_p, row3, col3)

    z0, z1s_p = pl.pallas_call(
        _tc2_body,
        out_shape=(
            jax.ShapeDtypeStruct((n, D_HID), jnp.float32),
            jax.ShapeDtypeStruct((NP, D_HID), jnp.float32),
        ),
    )(y0, agg1, dis, b1.reshape(1, D_HID), W2_0, W2_1)

    agg2 = _sc_scatter(z1s_p, row3, col3)

    out = pl.pallas_call(
        _tc3_body,
        out_shape=jax.ShapeDtypeStruct((n, D_HID), jnp.float32),
    )(z0, agg2, dis, b2.reshape(1, D_HID))
    return out
